# baseline (speedup 1.0000x reference)
import jax
import jax.numpy as jnp
from jax.experimental import pallas as pl


def _sage(x, ei, Wl, bl, Wr):
    src = ei[0]
    dst = ei[1]
    m = jnp.take(x, src, axis=0)
    n = x.shape[0]
    s = jax.ops.segment_sum(m, dst, num_segments=n)
    c = jax.ops.segment_sum(jnp.ones((m.shape[0],), jnp.float32), dst, num_segments=n)
    mean = s / jnp.maximum(c, 1.0)[:, None]
    return mean @ Wl + bl + x @ Wr


def _gmp(x, b, G):
    s = jax.ops.segment_sum(x, b, num_segments=G)
    c = jax.ops.segment_sum(jnp.ones((x.shape[0],), jnp.float32), b, num_segments=G)
    return s / jnp.maximum(c, 1.0)[:, None]


def _final_kernel(rep_ref, w_ref, b_ref, o_ref):
    o_ref[...] = rep_ref[...] @ w_ref[...] + b_ref[...]


def kernel(x_void, x_vessel, edge_index_void, edge_index_vessel, batch_void, batch_vessel,
           Wl0_void, bl0_void, Wr0_void, Wl0_vessel, bl0_vessel, Wr0_vessel,
           Wl1_void, bl1_void, Wr1_void, Wl1_vessel, bl1_vessel, Wr1_vessel,
           lin_W, lin_b):
    G = 64
    h_void = jax.nn.relu(_sage(x_void, edge_index_void, Wl0_void, bl0_void, Wr0_void))
    h_vessel = jax.nn.relu(_sage(x_vessel, edge_index_vessel, Wl0_vessel, bl0_vessel, Wr0_vessel))
    h_void = jax.nn.relu(_sage(h_void, edge_index_void, Wl1_void, bl1_void, Wr1_void))
    h_vessel = jax.nn.relu(_sage(h_vessel, edge_index_vessel, Wl1_vessel, bl1_vessel, Wr1_vessel))
    rep = jnp.concatenate([_gmp(h_void, batch_void, G), _gmp(h_vessel, batch_vessel, G)], axis=1)
    out = pl.pallas_call(
        _final_kernel,
        out_shape=jax.ShapeDtypeStruct((G, lin_W.shape[1]), jnp.float32),
    )(rep, lin_W, lin_b[None, :])
    return out


# speedup vs baseline: 3.7024x; 3.7024x over previous
import functools

import jax
import jax.numpy as jnp
from jax import lax
from jax.experimental import pallas as pl
from jax.experimental.pallas import tpu as pltpu
from jax.experimental.pallas import tpu_sc as plsc

N = 50000
E = 800000
D = 128
G = 64
CH = 32
NCH = 4
NSUB = 16
RW = 128
RPS = 392
EPAD = NSUB * RPS * RW
ROWS_TOT = EPAD // RW
K = 4
NB = RPS // K
NACC = 50048
ZR = NACC // NSUB
ZR_LAST = N - (NSUB - 1) * ZR
BN = 400
GRID = N // BN


def _seg_chunk(src2d, dst2d, xref, out, acc, sbuf, dbuf, rows, zsem, gsem, ssem,
               s, zrow, ch):
    pltpu.sync_copy(zrow, acc.at[pl.ds(s * ZR, ZR)])
    plsc.subcore_barrier()

    def body(b, carry):
        base = s * RPS + b * K
        pltpu.sync_copy(src2d.at[pl.ds(base, K)], sbuf)
        pltpu.sync_copy(dst2d.at[pl.ds(base, K)], dbuf)
        gds = [pltpu.async_copy(xref.at[sbuf.at[j]],
                                rows.at[pl.ds(j * RW, RW)], gsem)
               for j in range(K)]
        for dsc in gds:
            dsc.wait()
        sds = [pltpu.async_copy(rows.at[pl.ds(j * RW, RW)],
                                acc.at[dbuf.at[j]], ssem, add=True)
               for j in range(K)]
        for dsc in sds:
            dsc.wait()
        return carry

    lax.fori_loop(0, NB, body, 0)
    plsc.subcore_barrier()

    @pl.when(s < NSUB - 1)
    def _():
        pltpu.sync_copy(acc.at[pl.ds(s * ZR, ZR)],
                        out.at[ch, pl.ds(s * ZR, ZR)])

    @pl.when(s == NSUB - 1)
    def _():
        pltpu.sync_copy(acc.at[pl.ds((NSUB - 1) * ZR, ZR_LAST)],
                        out.at[ch, pl.ds((NSUB - 1) * ZR, ZR_LAST)])


def _make_segsum():
    mesh = plsc.VectorSubcoreMesh(core_axis_name="c", subcore_axis_name="s")

    @functools.partial(
        pl.kernel,
        out_type=jax.ShapeDtypeStruct((NCH, N, CH), jnp.float32),
        mesh=mesh,
        compiler_params=pltpu.CompilerParams(use_tc_tiling_on_sc=False),
        scratch_types=[
            pltpu.VMEM_SHARED((NACC, CH), jnp.float32),
            pltpu.VMEM((K, RW), jnp.int32),
            pltpu.VMEM((K, RW), jnp.int32),
            pltpu.VMEM((K * RW, CH), jnp.float32),
            pltpu.SemaphoreType.DMA,
            pltpu.SemaphoreType.DMA,
            pltpu.SemaphoreType.DMA,
        ],
    )
    def seg(src2d, dst2d, xc0, xc1, xc2, xc3, zrow, out,
            acc, sbuf, dbuf, rows, zsem, gsem, ssem):
        c = lax.axis_index("c")
        s = lax.axis_index("s")
        xs = (xc0, xc1, xc2, xc3)
        for cc in (0, 1):
            @pl.when(c == cc)
            def _():
                for k in (0, 1):
                    ch = 2 * cc + k
                    _seg_chunk(src2d, dst2d, xs[ch], out, acc, sbuf, dbuf,
                               rows, zsem, gsem, ssem, s, zrow, ch)

    return seg


def _make_counts():
    mesh = plsc.VectorSubcoreMesh(core_axis_name="c", subcore_axis_name="s")

    @functools.partial(
        pl.kernel,
        out_type=jax.ShapeDtypeStruct((2, N, 16), jnp.float32),
        mesh=mesh,
        compiler_params=pltpu.CompilerParams(use_tc_tiling_on_sc=False),
        scratch_types=[
            pltpu.VMEM_SHARED((NACC, 16), jnp.float32),
            pltpu.VMEM((K, RW), jnp.int32),
            pltpu.VMEM((RW, 16), jnp.float32),
            pltpu.SemaphoreType.DMA,
        ],
    )
    def cnt(dv2d, ds2d, ones_h, zcnt, out, acc, dbuf, ones_v, csem):
        c = lax.axis_index("c")
        s = lax.axis_index("s")
        pltpu.sync_copy(ones_h, ones_v)
        pltpu.sync_copy(zcnt, acc.at[pl.ds(s * ZR, ZR)])
        plsc.subcore_barrier()
        for cc in (0, 1):
            @pl.when(c == cc)
            def _():
                dref = (dv2d, ds2d)[cc]

                def body(b, carry):
                    base = s * RPS + b * K
                    pltpu.sync_copy(dref.at[pl.ds(base, K)], dbuf)
                    sds = [pltpu.async_copy(ones_v, acc.at[dbuf.at[j]],
                                            csem, add=True)
                           for j in range(K)]
                    for dsc in sds:
                        dsc.wait()
                    return carry

                lax.fori_loop(0, NB, body, 0)
                plsc.subcore_barrier()

                @pl.when(s < NSUB - 1)
                def _w():
                    pltpu.sync_copy(acc.at[pl.ds(s * ZR, ZR)],
                                    out.at[cc, pl.ds(s * ZR, ZR)])

                @pl.when(s == NSUB - 1)
                def _w2():
                    pltpu.sync_copy(acc.at[pl.ds((NSUB - 1) * ZR, ZR_LAST)],
                                    out.at[cc, pl.ds((NSUB - 1) * ZR, ZR_LAST)])

    return cnt


_segsum = _make_segsum()
_counts = _make_counts()


def _layer0_body(s4, x4, cnt, wl, wr, bl, h4):
    sm = jnp.concatenate([s4[c] for c in range(NCH)], axis=1)
    xm = jnp.concatenate([x4[c] for c in range(NCH)], axis=1)
    inv = 1.0 / jnp.maximum(cnt[:, 0:1], 1.0)
    h = jnp.maximum(
        jnp.dot(sm * inv, wl[...], preferred_element_type=jnp.float32)
        + bl[0:1, :]
        + jnp.dot(xm, wr[...], preferred_element_type=jnp.float32),
        0.0)
    for c in range(NCH):
        h4[c] = h[:, CH * c:CH * (c + 1)]


def _layer1_body(s4, x4, cnt, wl, wr, bl, batch, pool, pcnt):
    i = pl.program_id(0)
    sm = jnp.concatenate([s4[c] for c in range(NCH)], axis=1)
    xm = jnp.concatenate([x4[c] for c in range(NCH)], axis=1)
    inv = 1.0 / jnp.maximum(cnt[:, 0:1], 1.0)
    h = jnp.maximum(
        jnp.dot(sm * inv, wl[...], preferred_element_type=jnp.float32)
        + bl[0:1, :]
        + jnp.dot(xm, wr[...], preferred_element_type=jnp.float32),
        0.0)
    oneh = (batch[0, 0, :][:, None]
            == lax.broadcasted_iota(jnp.int32, (BN, G), 1)).astype(jnp.float32)
    contrib = lax.dot_general(oneh, h, (((0,), (0,)), ((), ())),
                              preferred_element_type=jnp.float32)
    ccontrib = jnp.broadcast_to(jnp.sum(oneh, axis=0)[:, None], (G, D))

    @pl.when(i == 0)
    def _():
        pool[...] = contrib
        pcnt[...] = ccontrib

    @pl.when(i > 0)
    def _():
        pool[...] += contrib
        pcnt[...] += ccontrib


def _head_body(pv, cv, ps, cs, w, b, o):
    a = pv[...] / jnp.maximum(cv[...], 1.0)
    bb = ps[...] / jnp.maximum(cs[...], 1.0)
    rep = jnp.concatenate([a, bb], axis=1)
    o[...] = jnp.dot(rep, w[...], preferred_element_type=jnp.float32) + b[0:1, :]


def _tc_layer0(s4, x4, cnt, wl, wr, bl2):
    return pl.pallas_call(
        _layer0_body,
        grid=(GRID,),
        in_specs=[
            pl.BlockSpec((NCH, BN, CH), lambda i: (0, i, 0)),
            pl.BlockSpec((NCH, BN, CH), lambda i: (0, i, 0)),
            pl.BlockSpec((BN, 16), lambda i: (i, 0)),
            pl.BlockSpec((D, D), lambda i: (0, 0)),
            pl.BlockSpec((D, D), lambda i: (0, 0)),
            pl.BlockSpec((8, D), lambda i: (0, 0)),
        ],
        out_specs=pl.BlockSpec((NCH, BN, CH), lambda i: (0, i, 0)),
        out_shape=jax.ShapeDtypeStruct((NCH, N, CH), jnp.float32),
    )(s4, x4, cnt, wl, wr, bl2)


def _tc_layer1(s4, x4, cnt, wl, wr, bl2, batch3):
    return pl.pallas_call(
        _layer1_body,
        grid=(GRID,),
        in_specs=[
            pl.BlockSpec((NCH, BN, CH), lambda i: (0, i, 0)),
            pl.BlockSpec((NCH, BN, CH), lambda i: (0, i, 0)),
            pl.BlockSpec((BN, 16), lambda i: (i, 0)),
            pl.BlockSpec((D, D), lambda i: (0, 0)),
            pl.BlockSpec((D, D), lambda i: (0, 0)),
            pl.BlockSpec((8, D), lambda i: (0, 0)),
            pl.BlockSpec((1, 1, BN), lambda i: (i, 0, 0)),
        ],
        out_specs=[
            pl.BlockSpec((G, D), lambda i: (0, 0)),
            pl.BlockSpec((G, D), lambda i: (0, 0)),
        ],
        out_shape=[
            jax.ShapeDtypeStruct((G, D), jnp.float32),
            jax.ShapeDtypeStruct((G, D), jnp.float32),
        ],
    )(s4, x4, cnt, wl, wr, bl2, batch3)


def kernel(x_void, x_vessel, edge_index_void, edge_index_vessel, batch_void, batch_vessel,
           Wl0_void, bl0_void, Wr0_void, Wl0_vessel, bl0_vessel, Wr0_vessel,
           Wl1_void, bl1_void, Wr1_void, Wl1_vessel, bl1_vessel, Wr1_vessel,
           lin_W, lin_b):
    pad = EPAD - E
    ar = jnp.arange(pad, dtype=jnp.int32)
    pad_src = ar % N
    pad_dst = N + (ar % 16)

    def prep_edges(ei):
        src = jnp.concatenate([ei[0], pad_src]).reshape(ROWS_TOT, RW)
        dst = jnp.concatenate([ei[1], pad_dst]).reshape(ROWS_TOT, RW)
        return src, dst

    src_v, dst_v = prep_edges(edge_index_void)
    src_s, dst_s = prep_edges(edge_index_vessel)

    zrow = jnp.zeros((ZR, CH), jnp.float32)
    zcnt = jnp.zeros((ZR, 16), jnp.float32)
    ones_h = jnp.ones((RW, 16), jnp.float32)

    def chunks(x):
        return [x[:, CH * c:CH * (c + 1)] for c in range(NCH)]

    xv_c = chunks(x_void)
    xs_c = chunks(x_vessel)
    xv4 = jnp.stack(xv_c)
    xs4 = jnp.stack(xs_c)
    b3_v = batch_void.reshape(GRID, 1, BN)
    b3_s = batch_vessel.reshape(GRID, 1, BN)

    cnt2 = _counts(dst_v, dst_s, ones_h, zcnt)
    cnt_v = cnt2[0]
    cnt_s = cnt2[1]

    s0_v = _segsum(src_v, dst_v, *xv_c, zrow)
    s0_s = _segsum(src_s, dst_s, *xs_c, zrow)
    h0_v = _tc_layer0(s0_v, xv4, cnt_v, Wl0_void, Wr0_void,
                      jnp.tile(bl0_void[None, :], (8, 1)))
    h0_s = _tc_layer0(s0_s, xs4, cnt_s, Wl0_vessel, Wr0_vessel,
                      jnp.tile(bl0_vessel[None, :], (8, 1)))

    s1_v = _segsum(src_v, dst_v, h0_v[0], h0_v[1], h0_v[2], h0_v[3], zrow)
    s1_s = _segsum(src_s, dst_s, h0_s[0], h0_s[1], h0_s[2], h0_s[3], zrow)
    pool_v, pcnt_v = _tc_layer1(s1_v, h0_v, cnt_v, Wl1_void, Wr1_void,
                                jnp.tile(bl1_void[None, :], (8, 1)), b3_v)
    pool_s, pcnt_s = _tc_layer1(s1_s, h0_s, cnt_s, Wl1_vessel, Wr1_vessel,
                                jnp.tile(bl1_vessel[None, :], (8, 1)), b3_s)

    out = pl.pallas_call(
        _head_body,
        out_shape=jax.ShapeDtypeStruct((G, lin_W.shape[1]), jnp.float32),
    )(pool_v, pcnt_v, pool_s, pcnt_s, lin_W,
      jnp.tile(lin_b[None, :], (8, 1)))
    return out


# speedup vs baseline: 4.6424x; 1.2539x over previous
import functools

import jax
import jax.numpy as jnp
from jax import lax
from jax.experimental import pallas as pl
from jax.experimental.pallas import tpu as pltpu
from jax.experimental.pallas import tpu_sc as plsc

N = 50000
E = 800000
D = 128
G = 64
CH = 32
NCH = 4
NSUB = 16
RW = 128
RPS = 392
EPAD = NSUB * RPS * RW
ROWS_TOT = EPAD // RW
K = 4
NB = RPS // K
KB = 2
NBL = RPS // KB
NACC = 50048
ZR = NACC // NSUB
ZR_LAST = N - (NSUB - 1) * ZR
BN = 400
GRID = N // BN


def _seg_chunk(src2d, dst2d, xref, out, acc, sbuf, dbuf, rows, isem, gsem, ssem,
               s, zrow, ch):
    pltpu.sync_copy(zrow, acc.at[pl.ds(s * ZR, ZR)])
    plsc.subcore_barrier()
    base0 = s * RPS

    def load_idx(i, p):
        a = pltpu.async_copy(src2d.at[pl.ds(base0 + i * KB, KB)],
                             sbuf.at[p], isem)
        b = pltpu.async_copy(dst2d.at[pl.ds(base0 + i * KB, KB)],
                             dbuf.at[p], isem)
        return a, b

    def drain_idx(p):
        pltpu.make_async_copy(src2d.at[pl.ds(0, KB)], sbuf.at[p], isem).wait()
        pltpu.make_async_copy(dst2d.at[pl.ds(0, KB)], dbuf.at[p], isem).wait()

    def fire_gathers(p):
        for j in range(KB):
            pltpu.async_copy(xref.at[sbuf.at[p, j]],
                             rows.at[p, pl.ds(j * RW, RW)], gsem)

    def drain_gathers(p):
        for j in range(KB):
            pltpu.make_async_copy(xref.at[pl.ds(0, RW)],
                                  rows.at[p, pl.ds(j * RW, RW)], gsem).wait()

    def fire_scatters(p):
        for j in range(KB):
            pltpu.async_copy(rows.at[p, pl.ds(j * RW, RW)],
                             acc.at[dbuf.at[p, j]], ssem, add=True)

    def drain_scatters(p):
        for j in range(KB):
            pltpu.make_async_copy(xref.at[pl.ds(0, RW)],
                                  rows.at[p, pl.ds(j * RW, RW)], ssem).wait()

    def step(i, p, next_gather, next_idx):
        q = 1 - p
        drain_gathers(p)
        fire_scatters(p)
        if next_gather:
            drain_idx(q)
            fire_gathers(q)
        drain_scatters(p)
        if next_idx:
            load_idx(i + 2, p)

    a, b = load_idx(0, 0)
    a.wait()
    b.wait()
    fire_gathers(0)
    load_idx(1, 1)

    def body(t, carry):
        i0 = 2 * t
        step(i0, 0, True, True)
        step(i0 + 1, 1, True, True)
        return carry

    lax.fori_loop(0, (NBL - 2) // 2, body, 0)
    step(NBL - 2, 0, True, False)
    step(NBL - 1, 1, False, False)
    plsc.subcore_barrier()

    @pl.when(s < NSUB - 1)
    def _():
        pltpu.sync_copy(acc.at[pl.ds(s * ZR, ZR)],
                        out.at[ch, pl.ds(s * ZR, ZR)])

    @pl.when(s == NSUB - 1)
    def _():
        pltpu.sync_copy(acc.at[pl.ds((NSUB - 1) * ZR, ZR_LAST)],
                        out.at[ch, pl.ds((NSUB - 1) * ZR, ZR_LAST)])


def _make_segsum():
    mesh = plsc.VectorSubcoreMesh(core_axis_name="c", subcore_axis_name="s")

    @functools.partial(
        pl.kernel,
        out_type=jax.ShapeDtypeStruct((NCH, N, CH), jnp.float32),
        mesh=mesh,
        compiler_params=pltpu.CompilerParams(use_tc_tiling_on_sc=False),
        scratch_types=[
            pltpu.VMEM_SHARED((NACC, CH), jnp.float32),
            pltpu.VMEM((2, KB, RW), jnp.int32),
            pltpu.VMEM((2, KB, RW), jnp.int32),
            pltpu.VMEM((2, KB * RW, CH), jnp.float32),
            pltpu.SemaphoreType.DMA,
            pltpu.SemaphoreType.DMA,
            pltpu.SemaphoreType.DMA,
        ],
    )
    def seg(src2d, dst2d, xc0, xc1, xc2, xc3, zrow, out,
            acc, sbuf, dbuf, rows, isem, gsem, ssem):
        c = lax.axis_index("c")
        s = lax.axis_index("s")
        xs = (xc0, xc1, xc2, xc3)
        for cc in (0, 1):
            @pl.when(c == cc)
            def _():
                for k in (0, 1):
                    ch = 2 * cc + k
                    _seg_chunk(src2d, dst2d, xs[ch], out, acc, sbuf, dbuf,
                               rows, isem, gsem, ssem, s, zrow, ch)

    return seg


def _make_counts():
    mesh = plsc.VectorSubcoreMesh(core_axis_name="c", subcore_axis_name="s")

    @functools.partial(
        pl.kernel,
        out_type=jax.ShapeDtypeStruct((2, N, 16), jnp.float32),
        mesh=mesh,
        compiler_params=pltpu.CompilerParams(use_tc_tiling_on_sc=False),
        scratch_types=[
            pltpu.VMEM_SHARED((NACC, 16), jnp.float32),
            pltpu.VMEM((K, RW), jnp.int32),
            pltpu.VMEM((RW, 16), jnp.float32),
            pltpu.SemaphoreType.DMA,
        ],
    )
    def cnt(dv2d, ds2d, ones_h, zcnt, out, acc, dbuf, ones_v, csem):
        c = lax.axis_index("c")
        s = lax.axis_index("s")
        pltpu.sync_copy(ones_h, ones_v)
        pltpu.sync_copy(zcnt, acc.at[pl.ds(s * ZR, ZR)])
        plsc.subcore_barrier()
        for cc in (0, 1):
            @pl.when(c == cc)
            def _():
                dref = (dv2d, ds2d)[cc]

                def body(b, carry):
                    base = s * RPS + b * K
                    pltpu.sync_copy(dref.at[pl.ds(base, K)], dbuf)
                    sds = [pltpu.async_copy(ones_v, acc.at[dbuf.at[j]],
                                            csem, add=True)
                           for j in range(K)]
                    for dsc in sds:
                        dsc.wait()
                    return carry

                lax.fori_loop(0, NB, body, 0)
                plsc.subcore_barrier()

                @pl.when(s < NSUB - 1)
                def _w():
                    pltpu.sync_copy(acc.at[pl.ds(s * ZR, ZR)],
                                    out.at[cc, pl.ds(s * ZR, ZR)])

                @pl.when(s == NSUB - 1)
                def _w2():
                    pltpu.sync_copy(acc.at[pl.ds((NSUB - 1) * ZR, ZR_LAST)],
                                    out.at[cc, pl.ds((NSUB - 1) * ZR, ZR_LAST)])

    return cnt


_segsum = _make_segsum()
_counts = _make_counts()


def _layer0_body(s4, x4, cnt, wl, wr, bl, h4):
    sm = jnp.concatenate([s4[c] for c in range(NCH)], axis=1)
    xm = jnp.concatenate([x4[c] for c in range(NCH)], axis=1)
    inv = 1.0 / jnp.maximum(cnt[:, 0:1], 1.0)
    h = jnp.maximum(
        jnp.dot(sm * inv, wl[...], preferred_element_type=jnp.float32)
        + bl[0:1, :]
        + jnp.dot(xm, wr[...], preferred_element_type=jnp.float32),
        0.0)
    for c in range(NCH):
        h4[c] = h[:, CH * c:CH * (c + 1)]


def _layer1_body(s4, x4, cnt, wl, wr, bl, batch, pool, pcnt):
    i = pl.program_id(0)
    sm = jnp.concatenate([s4[c] for c in range(NCH)], axis=1)
    xm = jnp.concatenate([x4[c] for c in range(NCH)], axis=1)
    inv = 1.0 / jnp.maximum(cnt[:, 0:1], 1.0)
    h = jnp.maximum(
        jnp.dot(sm * inv, wl[...], preferred_element_type=jnp.float32)
        + bl[0:1, :]
        + jnp.dot(xm, wr[...], preferred_element_type=jnp.float32),
        0.0)
    oneh = (batch[0, 0, :][:, None]
            == lax.broadcasted_iota(jnp.int32, (BN, G), 1)).astype(jnp.float32)
    contrib = lax.dot_general(oneh, h, (((0,), (0,)), ((), ())),
                              preferred_element_type=jnp.float32)
    ccontrib = jnp.broadcast_to(jnp.sum(oneh, axis=0)[:, None], (G, D))

    @pl.when(i == 0)
    def _():
        pool[...] = contrib
        pcnt[...] = ccontrib

    @pl.when(i > 0)
    def _():
        pool[...] += contrib
        pcnt[...] += ccontrib


def _head_body(pv, cv, ps, cs, w, b, o):
    a = pv[...] / jnp.maximum(cv[...], 1.0)
    bb = ps[...] / jnp.maximum(cs[...], 1.0)
    rep = jnp.concatenate([a, bb], axis=1)
    o[...] = jnp.dot(rep, w[...], preferred_element_type=jnp.float32) + b[0:1, :]


def _tc_layer0(s4, x4, cnt, wl, wr, bl2):
    return pl.pallas_call(
        _layer0_body,
        grid=(GRID,),
        in_specs=[
            pl.BlockSpec((NCH, BN, CH), lambda i: (0, i, 0)),
            pl.BlockSpec((NCH, BN, CH), lambda i: (0, i, 0)),
            pl.BlockSpec((BN, 16), lambda i: (i, 0)),
            pl.BlockSpec((D, D), lambda i: (0, 0)),
            pl.BlockSpec((D, D), lambda i: (0, 0)),
            pl.BlockSpec((8, D), lambda i: (0, 0)),
        ],
        out_specs=pl.BlockSpec((NCH, BN, CH), lambda i: (0, i, 0)),
        out_shape=jax.ShapeDtypeStruct((NCH, N, CH), jnp.float32),
    )(s4, x4, cnt, wl, wr, bl2)


def _tc_layer1(s4, x4, cnt, wl, wr, bl2, batch3):
    return pl.pallas_call(
        _layer1_body,
        grid=(GRID,),
        in_specs=[
            pl.BlockSpec((NCH, BN, CH), lambda i: (0, i, 0)),
            pl.BlockSpec((NCH, BN, CH), lambda i: (0, i, 0)),
            pl.BlockSpec((BN, 16), lambda i: (i, 0)),
            pl.BlockSpec((D, D), lambda i: (0, 0)),
            pl.BlockSpec((D, D), lambda i: (0, 0)),
            pl.BlockSpec((8, D), lambda i: (0, 0)),
            pl.BlockSpec((1, 1, BN), lambda i: (i, 0, 0)),
        ],
        out_specs=[
            pl.BlockSpec((G, D), lambda i: (0, 0)),
            pl.BlockSpec((G, D), lambda i: (0, 0)),
        ],
        out_shape=[
            jax.ShapeDtypeStruct((G, D), jnp.float32),
            jax.ShapeDtypeStruct((G, D), jnp.float32),
        ],
    )(s4, x4, cnt, wl, wr, bl2, batch3)


def kernel(x_void, x_vessel, edge_index_void, edge_index_vessel, batch_void, batch_vessel,
           Wl0_void, bl0_void, Wr0_void, Wl0_vessel, bl0_vessel, Wr0_vessel,
           Wl1_void, bl1_void, Wr1_void, Wl1_vessel, bl1_vessel, Wr1_vessel,
           lin_W, lin_b):
    pad = EPAD - E
    ar = jnp.arange(pad, dtype=jnp.int32)
    pad_src = ar % N
    pad_dst = N + (ar % 16)

    def prep_edges(ei):
        src = jnp.concatenate([ei[0], pad_src]).reshape(ROWS_TOT, RW)
        dst = jnp.concatenate([ei[1], pad_dst]).reshape(ROWS_TOT, RW)
        return src, dst

    src_v, dst_v = prep_edges(edge_index_void)
    src_s, dst_s = prep_edges(edge_index_vessel)

    zrow = jnp.zeros((ZR, CH), jnp.float32)
    zcnt = jnp.zeros((ZR, 16), jnp.float32)
    ones_h = jnp.ones((RW, 16), jnp.float32)

    def chunks(x):
        return [x[:, CH * c:CH * (c + 1)] for c in range(NCH)]

    xv_c = chunks(x_void)
    xs_c = chunks(x_vessel)
    xv4 = jnp.stack(xv_c)
    xs4 = jnp.stack(xs_c)
    b3_v = batch_void.reshape(GRID, 1, BN)
    b3_s = batch_vessel.reshape(GRID, 1, BN)

    cnt2 = _counts(dst_v, dst_s, ones_h, zcnt)
    cnt_v = cnt2[0]
    cnt_s = cnt2[1]

    s0_v = _segsum(src_v, dst_v, *xv_c, zrow)
    s0_s = _segsum(src_s, dst_s, *xs_c, zrow)
    h0_v = _tc_layer0(s0_v, xv4, cnt_v, Wl0_void, Wr0_void,
                      jnp.tile(bl0_void[None, :], (8, 1)))
    h0_s = _tc_layer0(s0_s, xs4, cnt_s, Wl0_vessel, Wr0_vessel,
                      jnp.tile(bl0_vessel[None, :], (8, 1)))

    s1_v = _segsum(src_v, dst_v, h0_v[0], h0_v[1], h0_v[2], h0_v[3], zrow)
    s1_s = _segsum(src_s, dst_s, h0_s[0], h0_s[1], h0_s[2], h0_s[3], zrow)
    pool_v, pcnt_v = _tc_layer1(s1_v, h0_v, cnt_v, Wl1_void, Wr1_void,
                                jnp.tile(bl1_void[None, :], (8, 1)), b3_v)
    pool_s, pcnt_s = _tc_layer1(s1_s, h0_s, cnt_s, Wl1_vessel, Wr1_vessel,
                                jnp.tile(bl1_vessel[None, :], (8, 1)), b3_s)

    out = pl.pallas_call(
        _head_body,
        out_shape=jax.ShapeDtypeStruct((G, lin_W.shape[1]), jnp.float32),
    )(pool_v, pcnt_v, pool_s, pcnt_s, lin_W,
      jnp.tile(lin_b[None, :], (8, 1)))
    return out


# speedup vs baseline: 4.6477x; 1.0011x over previous
import functools

import jax
import jax.numpy as jnp
from jax import lax
from jax.experimental import pallas as pl
from jax.experimental.pallas import tpu as pltpu
from jax.experimental.pallas import tpu_sc as plsc

N = 50000
E = 800000
D = 128
G = 64
CH = 32
NCH = 4
NSUB = 16
RW = 256
RPS = 196
EPAD = NSUB * RPS * RW
ROWS_TOT = EPAD // RW
K = 2
NB = RPS // K
KB = 1
NBL = RPS // KB
NACC = 50048
ZR = NACC // NSUB
ZR_LAST = N - (NSUB - 1) * ZR
BN = 400
GRID = N // BN


def _seg_chunk(src2d, dst2d, xref, out, acc, sbuf, dbuf, rows, isem, gsem, ssem,
               s, zrow, ch):
    pltpu.sync_copy(zrow, acc.at[pl.ds(s * ZR, ZR)])
    plsc.subcore_barrier()
    base0 = s * RPS

    def load_idx(i, p):
        a = pltpu.async_copy(src2d.at[pl.ds(base0 + i * KB, KB)],
                             sbuf.at[p], isem)
        b = pltpu.async_copy(dst2d.at[pl.ds(base0 + i * KB, KB)],
                             dbuf.at[p], isem)
        return a, b

    def drain_idx(p):
        pltpu.make_async_copy(src2d.at[pl.ds(0, KB)], sbuf.at[p], isem).wait()
        pltpu.make_async_copy(dst2d.at[pl.ds(0, KB)], dbuf.at[p], isem).wait()

    def fire_gathers(p):
        for j in range(KB):
            pltpu.async_copy(xref.at[sbuf.at[p, j]],
                             rows.at[p, pl.ds(j * RW, RW)], gsem)

    def drain_gathers(p):
        for j in range(KB):
            pltpu.make_async_copy(xref.at[pl.ds(0, RW)],
                                  rows.at[p, pl.ds(j * RW, RW)], gsem).wait()

    def fire_scatters(p):
        for j in range(KB):
            pltpu.async_copy(rows.at[p, pl.ds(j * RW, RW)],
                             acc.at[dbuf.at[p, j]], ssem, add=True)

    def drain_scatters(p):
        for j in range(KB):
            pltpu.make_async_copy(xref.at[pl.ds(0, RW)],
                                  rows.at[p, pl.ds(j * RW, RW)], ssem).wait()

    def step(i, p, next_gather, next_idx):
        q = 1 - p
        drain_gathers(p)
        fire_scatters(p)
        if next_gather:
            drain_idx(q)
            fire_gathers(q)
        drain_scatters(p)
        if next_idx:
            load_idx(i + 2, p)

    a, b = load_idx(0, 0)
    a.wait()
    b.wait()
    fire_gathers(0)
    load_idx(1, 1)

    def body(t, carry):
        i0 = 2 * t
        step(i0, 0, True, True)
        step(i0 + 1, 1, True, True)
        return carry

    lax.fori_loop(0, (NBL - 2) // 2, body, 0)
    step(NBL - 2, 0, True, False)
    step(NBL - 1, 1, False, False)
    plsc.subcore_barrier()

    @pl.when(s < NSUB - 1)
    def _():
        pltpu.sync_copy(acc.at[pl.ds(s * ZR, ZR)],
                        out.at[ch, pl.ds(s * ZR, ZR)])

    @pl.when(s == NSUB - 1)
    def _():
        pltpu.sync_copy(acc.at[pl.ds((NSUB - 1) * ZR, ZR_LAST)],
                        out.at[ch, pl.ds((NSUB - 1) * ZR, ZR_LAST)])


def _make_segsum():
    mesh = plsc.VectorSubcoreMesh(core_axis_name="c", subcore_axis_name="s")

    @functools.partial(
        pl.kernel,
        out_type=jax.ShapeDtypeStruct((NCH, N, CH), jnp.float32),
        mesh=mesh,
        compiler_params=pltpu.CompilerParams(use_tc_tiling_on_sc=False),
        scratch_types=[
            pltpu.VMEM_SHARED((NACC, CH), jnp.float32),
            pltpu.VMEM((2, KB, RW), jnp.int32),
            pltpu.VMEM((2, KB, RW), jnp.int32),
            pltpu.VMEM((2, KB * RW, CH), jnp.float32),
            pltpu.SemaphoreType.DMA,
            pltpu.SemaphoreType.DMA,
            pltpu.SemaphoreType.DMA,
        ],
    )
    def seg(src2d, dst2d, xc0, xc1, xc2, xc3, zrow, out,
            acc, sbuf, dbuf, rows, isem, gsem, ssem):
        c = lax.axis_index("c")
        s = lax.axis_index("s")
        xs = (xc0, xc1, xc2, xc3)
        for cc in (0, 1):
            @pl.when(c == cc)
            def _():
                for k in (0, 1):
                    ch = 2 * cc + k
                    _seg_chunk(src2d, dst2d, xs[ch], out, acc, sbuf, dbuf,
                               rows, isem, gsem, ssem, s, zrow, ch)

    return seg


def _make_counts():
    mesh = plsc.VectorSubcoreMesh(core_axis_name="c", subcore_axis_name="s")

    @functools.partial(
        pl.kernel,
        out_type=jax.ShapeDtypeStruct((2, N, 16), jnp.float32),
        mesh=mesh,
        compiler_params=pltpu.CompilerParams(use_tc_tiling_on_sc=False),
        scratch_types=[
            pltpu.VMEM_SHARED((NACC, 16), jnp.float32),
            pltpu.VMEM((K, RW), jnp.int32),
            pltpu.VMEM((RW, 16), jnp.float32),
            pltpu.SemaphoreType.DMA,
        ],
    )
    def cnt(dv2d, ds2d, ones_h, zcnt, out, acc, dbuf, ones_v, csem):
        c = lax.axis_index("c")
        s = lax.axis_index("s")
        pltpu.sync_copy(ones_h, ones_v)
        pltpu.sync_copy(zcnt, acc.at[pl.ds(s * ZR, ZR)])
        plsc.subcore_barrier()
        for cc in (0, 1):
            @pl.when(c == cc)
            def _():
                dref = (dv2d, ds2d)[cc]

                def body(b, carry):
                    base = s * RPS + b * K
                    pltpu.sync_copy(dref.at[pl.ds(base, K)], dbuf)
                    sds = [pltpu.async_copy(ones_v, acc.at[dbuf.at[j]],
                                            csem, add=True)
                           for j in range(K)]
                    for dsc in sds:
                        dsc.wait()
                    return carry

                lax.fori_loop(0, NB, body, 0)
                plsc.subcore_barrier()

                @pl.when(s < NSUB - 1)
                def _w():
                    pltpu.sync_copy(acc.at[pl.ds(s * ZR, ZR)],
                                    out.at[cc, pl.ds(s * ZR, ZR)])

                @pl.when(s == NSUB - 1)
                def _w2():
                    pltpu.sync_copy(acc.at[pl.ds((NSUB - 1) * ZR, ZR_LAST)],
                                    out.at[cc, pl.ds((NSUB - 1) * ZR, ZR_LAST)])

    return cnt


_segsum = _make_segsum()
_counts = _make_counts()


def _layer0_body(s4, x4, cnt, wl, wr, bl, h4):
    sm = jnp.concatenate([s4[c] for c in range(NCH)], axis=1)
    xm = jnp.concatenate([x4[c] for c in range(NCH)], axis=1)
    inv = 1.0 / jnp.maximum(cnt[:, 0:1], 1.0)
    h = jnp.maximum(
        jnp.dot(sm * inv, wl[...], preferred_element_type=jnp.float32)
        + bl[0:1, :]
        + jnp.dot(xm, wr[...], preferred_element_type=jnp.float32),
        0.0)
    for c in range(NCH):
        h4[c] = h[:, CH * c:CH * (c + 1)]


def _layer1_body(s4, x4, cnt, wl, wr, bl, batch, pool, pcnt):
    i = pl.program_id(0)
    sm = jnp.concatenate([s4[c] for c in range(NCH)], axis=1)
    xm = jnp.concatenate([x4[c] for c in range(NCH)], axis=1)
    inv = 1.0 / jnp.maximum(cnt[:, 0:1], 1.0)
    h = jnp.maximum(
        jnp.dot(sm * inv, wl[...], preferred_element_type=jnp.float32)
        + bl[0:1, :]
        + jnp.dot(xm, wr[...], preferred_element_type=jnp.float32),
        0.0)
    oneh = (batch[0, 0, :][:, None]
            == lax.broadcasted_iota(jnp.int32, (BN, G), 1)).astype(jnp.float32)
    contrib = lax.dot_general(oneh, h, (((0,), (0,)), ((), ())),
                              preferred_element_type=jnp.float32)
    ccontrib = jnp.broadcast_to(jnp.sum(oneh, axis=0)[:, None], (G, D))

    @pl.when(i == 0)
    def _():
        pool[...] = contrib
        pcnt[...] = ccontrib

    @pl.when(i > 0)
    def _():
        pool[...] += contrib
        pcnt[...] += ccontrib


def _head_body(pv, cv, ps, cs, w, b, o):
    a = pv[...] / jnp.maximum(cv[...], 1.0)
    bb = ps[...] / jnp.maximum(cs[...], 1.0)
    rep = jnp.concatenate([a, bb], axis=1)
    o[...] = jnp.dot(rep, w[...], preferred_element_type=jnp.float32) + b[0:1, :]


def _tc_layer0(s4, x4, cnt, wl, wr, bl2):
    return pl.pallas_call(
        _layer0_body,
        grid=(GRID,),
        in_specs=[
            pl.BlockSpec((NCH, BN, CH), lambda i: (0, i, 0)),
            pl.BlockSpec((NCH, BN, CH), lambda i: (0, i, 0)),
            pl.BlockSpec((BN, 16), lambda i: (i, 0)),
            pl.BlockSpec((D, D), lambda i: (0, 0)),
            pl.BlockSpec((D, D), lambda i: (0, 0)),
            pl.BlockSpec((8, D), lambda i: (0, 0)),
        ],
        out_specs=pl.BlockSpec((NCH, BN, CH), lambda i: (0, i, 0)),
        out_shape=jax.ShapeDtypeStruct((NCH, N, CH), jnp.float32),
    )(s4, x4, cnt, wl, wr, bl2)


def _tc_layer1(s4, x4, cnt, wl, wr, bl2, batch3):
    return pl.pallas_call(
        _layer1_body,
        grid=(GRID,),
        in_specs=[
            pl.BlockSpec((NCH, BN, CH), lambda i: (0, i, 0)),
            pl.BlockSpec((NCH, BN, CH), lambda i: (0, i, 0)),
            pl.BlockSpec((BN, 16), lambda i: (i, 0)),
            pl.BlockSpec((D, D), lambda i: (0, 0)),
            pl.BlockSpec((D, D), lambda i: (0, 0)),
            pl.BlockSpec((8, D), lambda i: (0, 0)),
            pl.BlockSpec((1, 1, BN), lambda i: (i, 0, 0)),
        ],
        out_specs=[
            pl.BlockSpec((G, D), lambda i: (0, 0)),
            pl.BlockSpec((G, D), lambda i: (0, 0)),
        ],
        out_shape=[
            jax.ShapeDtypeStruct((G, D), jnp.float32),
            jax.ShapeDtypeStruct((G, D), jnp.float32),
        ],
    )(s4, x4, cnt, wl, wr, bl2, batch3)


def kernel(x_void, x_vessel, edge_index_void, edge_index_vessel, batch_void, batch_vessel,
           Wl0_void, bl0_void, Wr0_void, Wl0_vessel, bl0_vessel, Wr0_vessel,
           Wl1_void, bl1_void, Wr1_void, Wl1_vessel, bl1_vessel, Wr1_vessel,
           lin_W, lin_b):
    pad = EPAD - E
    ar = jnp.arange(pad, dtype=jnp.int32)
    pad_src = ar % N
    pad_dst = N + (ar % 16)

    def prep_edges(ei):
        src = jnp.concatenate([ei[0], pad_src]).reshape(ROWS_TOT, RW)
        dst = jnp.concatenate([ei[1], pad_dst]).reshape(ROWS_TOT, RW)
        return src, dst

    src_v, dst_v = prep_edges(edge_index_void)
    src_s, dst_s = prep_edges(edge_index_vessel)

    zrow = jnp.zeros((ZR, CH), jnp.float32)
    zcnt = jnp.zeros((ZR, 16), jnp.float32)
    ones_h = jnp.ones((RW, 16), jnp.float32)

    def chunks(x):
        return [x[:, CH * c:CH * (c + 1)] for c in range(NCH)]

    xv_c = chunks(x_void)
    xs_c = chunks(x_vessel)
    xv4 = jnp.stack(xv_c)
    xs4 = jnp.stack(xs_c)
    b3_v = batch_void.reshape(GRID, 1, BN)
    b3_s = batch_vessel.reshape(GRID, 1, BN)

    cnt2 = _counts(dst_v, dst_s, ones_h, zcnt)
    cnt_v = cnt2[0]
    cnt_s = cnt2[1]

    s0_v = _segsum(src_v, dst_v, *xv_c, zrow)
    s0_s = _segsum(src_s, dst_s, *xs_c, zrow)
    h0_v = _tc_layer0(s0_v, xv4, cnt_v, Wl0_void, Wr0_void,
                      jnp.tile(bl0_void[None, :], (8, 1)))
    h0_s = _tc_layer0(s0_s, xs4, cnt_s, Wl0_vessel, Wr0_vessel,
                      jnp.tile(bl0_vessel[None, :], (8, 1)))

    s1_v = _segsum(src_v, dst_v, h0_v[0], h0_v[1], h0_v[2], h0_v[3], zrow)
    s1_s = _segsum(src_s, dst_s, h0_s[0], h0_s[1], h0_s[2], h0_s[3], zrow)
    pool_v, pcnt_v = _tc_layer1(s1_v, h0_v, cnt_v, Wl1_void, Wr1_void,
                                jnp.tile(bl1_void[None, :], (8, 1)), b3_v)
    pool_s, pcnt_s = _tc_layer1(s1_s, h0_s, cnt_s, Wl1_vessel, Wr1_vessel,
                                jnp.tile(bl1_vessel[None, :], (8, 1)), b3_s)

    out = pl.pallas_call(
        _head_body,
        out_shape=jax.ShapeDtypeStruct((G, lin_W.shape[1]), jnp.float32),
    )(pool_v, pcnt_v, pool_s, pcnt_s, lin_W,
      jnp.tile(lin_b[None, :], (8, 1)))
    return out


# speedup vs baseline: 5.6898x; 1.2242x over previous
import functools

import jax
import jax.numpy as jnp
from jax import lax
from jax.experimental import pallas as pl
from jax.experimental.pallas import tpu as pltpu
from jax.experimental.pallas import tpu_sc as plsc

N = 50000
E = 800000
D = 128
G = 64
CH = 32
NCH = 4
NSUB = 16
RW = 256
RPS = 196
EPAD = NSUB * RPS * RW
ROWS_TOT = EPAD // RW
K = 2
NB = RPS // K
NBL = RPS
NACC = 50048
ZR = NACC // NSUB
ZR_LAST = N - (NSUB - 1) * ZR
BN = 400
GRID = N // BN


def _seg_chunk(src2d, dst2d, xflat, out, acc, sbuf, dbuf, rows,
               isem, gsem, ssem, s, zrow, ch):
    pltpu.sync_copy(zrow, acc.at[pl.ds(s * ZR, ZR)])
    plsc.subcore_barrier()
    base0 = s * RPS

    def load_idx(i, p):
        a = pltpu.async_copy(src2d.at[pl.ds(base0 + i, 1)], sbuf.at[p], isem)
        b = pltpu.async_copy(dst2d.at[pl.ds(base0 + i, 1)], dbuf.at[p], isem)
        return a, b

    def drain_idx(p):
        pltpu.make_async_copy(src2d.at[pl.ds(0, 1)], sbuf.at[p], isem).wait()
        pltpu.make_async_copy(dst2d.at[pl.ds(0, 1)], dbuf.at[p], isem).wait()

    def fire_gather(p):
        pltpu.async_copy(xflat.at[sbuf.at[p, 0]], rows.at[p], gsem)

    def drain_gather(p):
        pltpu.make_async_copy(xflat.at[pl.ds(0, RW)], rows.at[p], gsem).wait()

    def fire_scatter(p):
        pltpu.async_copy(rows.at[p], acc.at[dbuf.at[p, 0]], ssem, add=True)

    def drain_scatter(p):
        pltpu.make_async_copy(xflat.at[pl.ds(0, RW)], rows.at[p], ssem).wait()

    def step(i, p, next_gather, next_idx):
        q = 1 - p
        drain_gather(p)
        fire_scatter(p)
        if next_gather:
            drain_idx(q)
            fire_gather(q)
        drain_scatter(p)
        if next_idx:
            load_idx(i + 2, p)

    a, b = load_idx(0, 0)
    a.wait()
    b.wait()
    fire_gather(0)
    load_idx(1, 1)

    def body(t, carry):
        i0 = 2 * t
        step(i0, 0, True, True)
        step(i0 + 1, 1, True, True)
        return carry

    lax.fori_loop(0, (NBL - 2) // 2, body, 0)
    step(NBL - 2, 0, True, False)
    step(NBL - 1, 1, False, False)
    plsc.subcore_barrier()

    @pl.when(s < NSUB - 1)
    def _():
        pltpu.sync_copy(acc.at[pl.ds(s * ZR, ZR)],
                        out.at[pl.ds(s * ZR, ZR), pl.ds(CH * ch, CH)])

    @pl.when(s == NSUB - 1)
    def _():
        pltpu.sync_copy(acc.at[pl.ds((NSUB - 1) * ZR, ZR_LAST)],
                        out.at[pl.ds((NSUB - 1) * ZR, ZR_LAST),
                               pl.ds(CH * ch, CH)])


def _make_segsum():
    mesh = plsc.VectorSubcoreMesh(core_axis_name="c", subcore_axis_name="s")

    @functools.partial(
        pl.kernel,
        out_type=[jax.ShapeDtypeStruct((N, D), jnp.float32),
                  jax.ShapeDtypeStruct((N, D), jnp.float32)],
        mesh=mesh,
        compiler_params=pltpu.CompilerParams(use_tc_tiling_on_sc=False),
        scratch_types=[
            pltpu.VMEM_SHARED((NACC, CH), jnp.float32),
            pltpu.VMEM((2, 1, RW), jnp.int32),
            pltpu.VMEM((2, 1, RW), jnp.int32),
            pltpu.VMEM((2, RW, CH), jnp.float32),
            pltpu.SemaphoreType.DMA,
            pltpu.SemaphoreType.DMA,
            pltpu.SemaphoreType.DMA,
        ],
    )
    def seg(sv0, sv1, sv2, sv3, ss0, ss1, ss2, ss3, dst_v, dst_s,
            xf_v, xf_s, zrow, out_v, out_s,
            acc, sbuf, dbuf, rows, isem, gsem, ssem):
        c = lax.axis_index("c")
        s = lax.axis_index("s")
        for cc in (0, 1):
            @pl.when(c == cc)
            def _():
                srcs = (sv0, sv1, sv2, sv3) if cc == 0 else (ss0, ss1, ss2, ss3)
                dst2d = dst_v if cc == 0 else dst_s
                xflat = xf_v if cc == 0 else xf_s
                out = out_v if cc == 0 else out_s
                for ch in range(NCH):
                    _seg_chunk(srcs[ch], dst2d, xflat, out, acc, sbuf, dbuf,
                               rows, isem, gsem, ssem, s, zrow, ch)

    return seg


def _make_counts():
    mesh = plsc.VectorSubcoreMesh(core_axis_name="c", subcore_axis_name="s")

    @functools.partial(
        pl.kernel,
        out_type=[jax.ShapeDtypeStruct((N, 16), jnp.float32),
                  jax.ShapeDtypeStruct((N, 16), jnp.float32)],
        mesh=mesh,
        compiler_params=pltpu.CompilerParams(use_tc_tiling_on_sc=False),
        scratch_types=[
            pltpu.VMEM_SHARED((NACC, 16), jnp.float32),
            pltpu.VMEM((K, RW), jnp.int32),
            pltpu.VMEM((RW, 16), jnp.float32),
            pltpu.SemaphoreType.DMA,
        ],
    )
    def cnt(dv2d, ds2d, ones_h, zcnt, out_v, out_s, acc, dbuf, ones_v, csem):
        c = lax.axis_index("c")
        s = lax.axis_index("s")
        pltpu.sync_copy(ones_h, ones_v)
        pltpu.sync_copy(zcnt, acc.at[pl.ds(s * ZR, ZR)])
        plsc.subcore_barrier()
        for cc in (0, 1):
            @pl.when(c == cc)
            def _():
                dref = (dv2d, ds2d)[cc]
                out = (out_v, out_s)[cc]

                def body(b, carry):
                    base = s * RPS + b * K
                    pltpu.sync_copy(dref.at[pl.ds(base, K)], dbuf)
                    sds = [pltpu.async_copy(ones_v, acc.at[dbuf.at[j]],
                                            csem, add=True)
                           for j in range(K)]
                    for dsc in sds:
                        dsc.wait()
                    return carry

                lax.fori_loop(0, NB, body, 0)
                plsc.subcore_barrier()

                @pl.when(s < NSUB - 1)
                def _w():
                    pltpu.sync_copy(acc.at[pl.ds(s * ZR, ZR)],
                                    out.at[pl.ds(s * ZR, ZR)])

                @pl.when(s == NSUB - 1)
                def _w2():
                    pltpu.sync_copy(acc.at[pl.ds((NSUB - 1) * ZR, ZR_LAST)],
                                    out.at[pl.ds((NSUB - 1) * ZR, ZR_LAST)])

    return cnt


_segsum = _make_segsum()
_counts = _make_counts()


def _layer0_body(sref, xref, cnt, wl, wr, bl, href):
    inv = 1.0 / jnp.maximum(cnt[:, 0:1], 1.0)
    href[...] = jnp.maximum(
        jnp.dot(sref[...] * inv, wl[...], preferred_element_type=jnp.float32)
        + bl[0:1, :]
        + jnp.dot(xref[...], wr[...], preferred_element_type=jnp.float32),
        0.0)


def _layer1_body(sref, xref, cnt, wl, wr, bl, batch, pool, pcnt):
    i = pl.program_id(0)
    inv = 1.0 / jnp.maximum(cnt[:, 0:1], 1.0)
    h = jnp.maximum(
        jnp.dot(sref[...] * inv, wl[...], preferred_element_type=jnp.float32)
        + bl[0:1, :]
        + jnp.dot(xref[...], wr[...], preferred_element_type=jnp.float32),
        0.0)
    oneh = (batch[0, 0, :][:, None]
            == lax.broadcasted_iota(jnp.int32, (BN, G), 1)).astype(jnp.float32)
    contrib = lax.dot_general(oneh, h, (((0,), (0,)), ((), ())),
                              preferred_element_type=jnp.float32)
    ccontrib = jnp.broadcast_to(jnp.sum(oneh, axis=0)[:, None], (G, D))

    @pl.when(i == 0)
    def _():
        pool[...] = contrib
        pcnt[...] = ccontrib

    @pl.when(i > 0)
    def _():
        pool[...] += contrib
        pcnt[...] += ccontrib


def _head_body(pv, cv, ps, cs, w, b, o):
    a = pv[...] / jnp.maximum(cv[...], 1.0)
    bb = ps[...] / jnp.maximum(cs[...], 1.0)
    rep = jnp.concatenate([a, bb], axis=1)
    o[...] = jnp.dot(rep, w[...], preferred_element_type=jnp.float32) + b[0:1, :]


def _tc_layer0(s, x, cnt, wl, wr, bl2):
    return pl.pallas_call(
        _layer0_body,
        grid=(GRID,),
        in_specs=[
            pl.BlockSpec((BN, D), lambda i: (i, 0)),
            pl.BlockSpec((BN, D), lambda i: (i, 0)),
            pl.BlockSpec((BN, 16), lambda i: (i, 0)),
            pl.BlockSpec((D, D), lambda i: (0, 0)),
            pl.BlockSpec((D, D), lambda i: (0, 0)),
            pl.BlockSpec((8, D), lambda i: (0, 0)),
        ],
        out_specs=pl.BlockSpec((BN, D), lambda i: (i, 0)),
        out_shape=jax.ShapeDtypeStruct((N, D), jnp.float32),
    )(s, x, cnt, wl, wr, bl2)


def _tc_layer1(s, x, cnt, wl, wr, bl2, batch3):
    return pl.pallas_call(
        _layer1_body,
        grid=(GRID,),
        in_specs=[
            pl.BlockSpec((BN, D), lambda i: (i, 0)),
            pl.BlockSpec((BN, D), lambda i: (i, 0)),
            pl.BlockSpec((BN, 16), lambda i: (i, 0)),
            pl.BlockSpec((D, D), lambda i: (0, 0)),
            pl.BlockSpec((D, D), lambda i: (0, 0)),
            pl.BlockSpec((8, D), lambda i: (0, 0)),
            pl.BlockSpec((1, 1, BN), lambda i: (i, 0, 0)),
        ],
        out_specs=[
            pl.BlockSpec((G, D), lambda i: (0, 0)),
            pl.BlockSpec((G, D), lambda i: (0, 0)),
        ],
        out_shape=[
            jax.ShapeDtypeStruct((G, D), jnp.float32),
            jax.ShapeDtypeStruct((G, D), jnp.float32),
        ],
    )(s, x, cnt, wl, wr, bl2, batch3)


def kernel(x_void, x_vessel, edge_index_void, edge_index_vessel, batch_void, batch_vessel,
           Wl0_void, bl0_void, Wr0_void, Wl0_vessel, bl0_vessel, Wr0_vessel,
           Wl1_void, bl1_void, Wr1_void, Wl1_vessel, bl1_vessel, Wr1_vessel,
           lin_W, lin_b):
    pad = EPAD - E
    ar = jnp.arange(pad, dtype=jnp.int32)
    pad_src = ar % N
    pad_dst = N + (ar % 16)

    def prep_edges(ei):
        src = jnp.concatenate([ei[0], pad_src])
        srcs = [(src * 4 + ch).reshape(ROWS_TOT, RW) for ch in range(NCH)]
        dst = jnp.concatenate([ei[1], pad_dst]).reshape(ROWS_TOT, RW)
        return srcs, dst

    src_v, dst_v = prep_edges(edge_index_void)
    src_s, dst_s = prep_edges(edge_index_vessel)

    zrow = jnp.zeros((ZR, CH), jnp.float32)
    zcnt = jnp.zeros((ZR, 16), jnp.float32)
    ones_h = jnp.ones((RW, 16), jnp.float32)
    b3_v = batch_void.reshape(GRID, 1, BN)
    b3_s = batch_vessel.reshape(GRID, 1, BN)

    cnt_v, cnt_s = _counts(dst_v, dst_s, ones_h, zcnt)

    s0_v, s0_s = _segsum(*src_v, *src_s, dst_v, dst_s,
                         x_void.reshape(NCH * N, CH),
                         x_vessel.reshape(NCH * N, CH), zrow)
    h0_v = _tc_layer0(s0_v, x_void, cnt_v, Wl0_void, Wr0_void,
                      jnp.tile(bl0_void[None, :], (8, 1)))
    h0_s = _tc_layer0(s0_s, x_vessel, cnt_s, Wl0_vessel, Wr0_vessel,
                      jnp.tile(bl0_vessel[None, :], (8, 1)))

    s1_v, s1_s = _segsum(*src_v, *src_s, dst_v, dst_s,
                         h0_v.reshape(NCH * N, CH),
                         h0_s.reshape(NCH * N, CH), zrow)
    pool_v, pcnt_v = _tc_layer1(s1_v, h0_v, cnt_v, Wl1_void, Wr1_void,
                                jnp.tile(bl1_void[None, :], (8, 1)), b3_v)
    pool_s, pcnt_s = _tc_layer1(s1_s, h0_s, cnt_s, Wl1_vessel, Wr1_vessel,
                                jnp.tile(bl1_vessel[None, :], (8, 1)), b3_s)

    out = pl.pallas_call(
        _head_body,
        out_shape=jax.ShapeDtypeStruct((G, lin_W.shape[1]), jnp.float32),
    )(pool_v, pcnt_v, pool_s, pcnt_s, lin_W,
      jnp.tile(lin_b[None, :], (8, 1)))
    return out


# speedup vs baseline: 6.0824x; 1.0690x over previous
import functools

import jax
import jax.numpy as jnp
from jax import lax
from jax.experimental import pallas as pl
from jax.experimental.pallas import tpu as pltpu
from jax.experimental.pallas import tpu_sc as plsc

N = 50000
E = 800000
D = 128
G = 64
CH = 32
NCH = 4
NSUB = 16
RW = 256
RPS = 196
EPAD = NSUB * RPS * RW
ROWS_TOT = EPAD // RW
K = 2
NB = RPS // K
NBL = RPS
NACC = 50048
ZR = NACC // NSUB
ZR_LAST = N - (NSUB - 1) * ZR
BN = 1000
GRID = N // BN


def _seg_chunk(src2d, dst2d, xflat, out, acc, sbuf, sbuf4, dbuf, rows,
               isem, gsem, ssem, s, zrow, ch):
    pltpu.sync_copy(zrow, acc.at[pl.ds(s * ZR, ZR)])
    plsc.subcore_barrier()
    base0 = s * RPS

    def load_idx(i, p):
        a = pltpu.async_copy(src2d.at[pl.ds(base0 + i, 1)], sbuf.at[p], isem)
        b = pltpu.async_copy(dst2d.at[pl.ds(base0 + i, 1)], dbuf.at[p], isem)
        return a, b

    def drain_idx(p):
        pltpu.make_async_copy(src2d.at[pl.ds(0, 1)], sbuf.at[p], isem).wait()
        pltpu.make_async_copy(dst2d.at[pl.ds(0, 1)], dbuf.at[p], isem).wait()

    def fire_gather(p):
        for t in range(RW // 16):
            v = sbuf[p, 0, pl.ds(16 * t, 16)]
            sbuf4[p, 0, pl.ds(16 * t, 16)] = v * 4 + ch
        pltpu.async_copy(xflat.at[sbuf4.at[p, 0]], rows.at[p], gsem)

    def drain_gather(p):
        pltpu.make_async_copy(xflat.at[pl.ds(0, RW)], rows.at[p], gsem).wait()

    def fire_scatter(p):
        pltpu.async_copy(rows.at[p], acc.at[dbuf.at[p, 0]], ssem, add=True)

    def drain_scatter(p):
        pltpu.make_async_copy(xflat.at[pl.ds(0, RW)], rows.at[p], ssem).wait()

    def step(i, p, next_gather, next_idx):
        q = 1 - p
        drain_gather(p)
        fire_scatter(p)
        if next_gather:
            drain_idx(q)
            fire_gather(q)
        drain_scatter(p)
        if next_idx:
            load_idx(i + 2, p)

    a, b = load_idx(0, 0)
    a.wait()
    b.wait()
    fire_gather(0)
    load_idx(1, 1)

    def body(t, carry):
        i0 = 2 * t
        step(i0, 0, True, True)
        step(i0 + 1, 1, True, True)
        return carry

    lax.fori_loop(0, (NBL - 2) // 2, body, 0)
    step(NBL - 2, 0, True, False)
    step(NBL - 1, 1, False, False)
    plsc.subcore_barrier()

    @pl.when(s < NSUB - 1)
    def _():
        pltpu.sync_copy(acc.at[pl.ds(s * ZR, ZR)],
                        out.at[pl.ds(s * ZR, ZR), pl.ds(CH * ch, CH)])

    @pl.when(s == NSUB - 1)
    def _():
        pltpu.sync_copy(acc.at[pl.ds((NSUB - 1) * ZR, ZR_LAST)],
                        out.at[pl.ds((NSUB - 1) * ZR, ZR_LAST),
                               pl.ds(CH * ch, CH)])


def _make_segsum():
    mesh = plsc.VectorSubcoreMesh(core_axis_name="c", subcore_axis_name="s")

    @functools.partial(
        pl.kernel,
        out_type=[jax.ShapeDtypeStruct((N, D), jnp.float32),
                  jax.ShapeDtypeStruct((N, D), jnp.float32)],
        mesh=mesh,
        compiler_params=pltpu.CompilerParams(use_tc_tiling_on_sc=False),
        scratch_types=[
            pltpu.VMEM_SHARED((NACC, CH), jnp.float32),
            pltpu.VMEM((2, 1, RW), jnp.int32),
            pltpu.VMEM((2, 1, RW), jnp.int32),
            pltpu.VMEM((2, 1, RW), jnp.int32),
            pltpu.VMEM((2, RW, CH), jnp.float32),
            pltpu.SemaphoreType.DMA,
            pltpu.SemaphoreType.DMA,
            pltpu.SemaphoreType.DMA,
        ],
    )
    def seg(src_v, src_s, dst_v, dst_s, xf_v, xf_s, zrow, out_v, out_s,
            acc, sbuf, sbuf4, dbuf, rows, isem, gsem, ssem):
        c = lax.axis_index("c")
        s = lax.axis_index("s")
        for cc in (0, 1):
            @pl.when(c == cc)
            def _():
                src2d = src_v if cc == 0 else src_s
                dst2d = dst_v if cc == 0 else dst_s
                xflat = xf_v if cc == 0 else xf_s
                out = out_v if cc == 0 else out_s
                for ch in range(NCH):
                    _seg_chunk(src2d, dst2d, xflat, out, acc, sbuf, sbuf4,
                               dbuf, rows, isem, gsem, ssem, s, zrow, ch)

    return seg


def _make_counts():
    mesh = plsc.VectorSubcoreMesh(core_axis_name="c", subcore_axis_name="s")

    @functools.partial(
        pl.kernel,
        out_type=[jax.ShapeDtypeStruct((N, 16), jnp.float32),
                  jax.ShapeDtypeStruct((N, 16), jnp.float32)],
        mesh=mesh,
        compiler_params=pltpu.CompilerParams(use_tc_tiling_on_sc=False),
        scratch_types=[
            pltpu.VMEM_SHARED((NACC, 16), jnp.float32),
            pltpu.VMEM((K, RW), jnp.int32),
            pltpu.VMEM((RW, 16), jnp.float32),
            pltpu.SemaphoreType.DMA,
        ],
    )
    def cnt(dv2d, ds2d, ones_h, zcnt, out_v, out_s, acc, dbuf, ones_v, csem):
        c = lax.axis_index("c")
        s = lax.axis_index("s")
        pltpu.sync_copy(ones_h, ones_v)
        pltpu.sync_copy(zcnt, acc.at[pl.ds(s * ZR, ZR)])
        plsc.subcore_barrier()
        for cc in (0, 1):
            @pl.when(c == cc)
            def _():
                dref = (dv2d, ds2d)[cc]
                out = (out_v, out_s)[cc]

                def body(b, carry):
                    base = s * RPS + b * K
                    pltpu.sync_copy(dref.at[pl.ds(base, K)], dbuf)
                    sds = [pltpu.async_copy(ones_v, acc.at[dbuf.at[j]],
                                            csem, add=True)
                           for j in range(K)]
                    for dsc in sds:
                        dsc.wait()
                    return carry

                lax.fori_loop(0, NB, body, 0)
                plsc.subcore_barrier()

                @pl.when(s < NSUB - 1)
                def _w():
                    pltpu.sync_copy(acc.at[pl.ds(s * ZR, ZR)],
                                    out.at[pl.ds(s * ZR, ZR)])

                @pl.when(s == NSUB - 1)
                def _w2():
                    pltpu.sync_copy(acc.at[pl.ds((NSUB - 1) * ZR, ZR_LAST)],
                                    out.at[pl.ds((NSUB - 1) * ZR, ZR_LAST)])

    return cnt


_segsum = _make_segsum()
_counts = _make_counts()


def _layer0_body(sref, xref, cnt, wl, wr, bl, href):
    inv = 1.0 / jnp.maximum(cnt[:, 0:1], 1.0)
    href[...] = jnp.maximum(
        jnp.dot(sref[...] * inv, wl[...], preferred_element_type=jnp.float32)
        + bl[0:1, :]
        + jnp.dot(xref[...], wr[...], preferred_element_type=jnp.float32),
        0.0)


def _layer1_body(sref, xref, cnt, wl, wr, bl, batch, pool, pcnt):
    i = pl.program_id(0)
    inv = 1.0 / jnp.maximum(cnt[:, 0:1], 1.0)
    h = jnp.maximum(
        jnp.dot(sref[...] * inv, wl[...], preferred_element_type=jnp.float32)
        + bl[0:1, :]
        + jnp.dot(xref[...], wr[...], preferred_element_type=jnp.float32),
        0.0)
    oneh = (batch[0, 0, :][:, None]
            == lax.broadcasted_iota(jnp.int32, (BN, G), 1)).astype(jnp.float32)
    contrib = lax.dot_general(oneh, h, (((0,), (0,)), ((), ())),
                              preferred_element_type=jnp.float32)
    ccontrib = jnp.broadcast_to(jnp.sum(oneh, axis=0)[:, None], (G, D))

    @pl.when(i == 0)
    def _():
        pool[...] = contrib
        pcnt[...] = ccontrib

    @pl.when(i > 0)
    def _():
        pool[...] += contrib
        pcnt[...] += ccontrib


def _head_body(pv, cv, ps, cs, w, b, o):
    a = pv[...] / jnp.maximum(cv[...], 1.0)
    bb = ps[...] / jnp.maximum(cs[...], 1.0)
    rep = jnp.concatenate([a, bb], axis=1)
    o[...] = jnp.dot(rep, w[...], preferred_element_type=jnp.float32) + b[0:1, :]


def _tc_layer0(s, x, cnt, wl, wr, bl2):
    return pl.pallas_call(
        _layer0_body,
        grid=(GRID,),
        in_specs=[
            pl.BlockSpec((BN, D), lambda i: (i, 0)),
            pl.BlockSpec((BN, D), lambda i: (i, 0)),
            pl.BlockSpec((BN, 16), lambda i: (i, 0)),
            pl.BlockSpec((D, D), lambda i: (0, 0)),
            pl.BlockSpec((D, D), lambda i: (0, 0)),
            pl.BlockSpec((8, D), lambda i: (0, 0)),
        ],
        out_specs=pl.BlockSpec((BN, D), lambda i: (i, 0)),
        out_shape=jax.ShapeDtypeStruct((N, D), jnp.float32),
    )(s, x, cnt, wl, wr, bl2)


def _tc_layer1(s, x, cnt, wl, wr, bl2, batch3):
    return pl.pallas_call(
        _layer1_body,
        grid=(GRID,),
        in_specs=[
            pl.BlockSpec((BN, D), lambda i: (i, 0)),
            pl.BlockSpec((BN, D), lambda i: (i, 0)),
            pl.BlockSpec((BN, 16), lambda i: (i, 0)),
            pl.BlockSpec((D, D), lambda i: (0, 0)),
            pl.BlockSpec((D, D), lambda i: (0, 0)),
            pl.BlockSpec((8, D), lambda i: (0, 0)),
            pl.BlockSpec((1, 1, BN), lambda i: (i, 0, 0)),
        ],
        out_specs=[
            pl.BlockSpec((G, D), lambda i: (0, 0)),
            pl.BlockSpec((G, D), lambda i: (0, 0)),
        ],
        out_shape=[
            jax.ShapeDtypeStruct((G, D), jnp.float32),
            jax.ShapeDtypeStruct((G, D), jnp.float32),
        ],
    )(s, x, cnt, wl, wr, bl2, batch3)


def kernel(x_void, x_vessel, edge_index_void, edge_index_vessel, batch_void, batch_vessel,
           Wl0_void, bl0_void, Wr0_void, Wl0_vessel, bl0_vessel, Wr0_vessel,
           Wl1_void, bl1_void, Wr1_void, Wl1_vessel, bl1_vessel, Wr1_vessel,
           lin_W, lin_b):
    pad = EPAD - E
    ar = jnp.arange(pad, dtype=jnp.int32)
    pad_src = ar % N
    pad_dst = N + (ar % 16)

    def prep_edges(ei):
        src = jnp.concatenate([ei[0], pad_src]).reshape(ROWS_TOT, RW)
        dst = jnp.concatenate([ei[1], pad_dst]).reshape(ROWS_TOT, RW)
        return src, dst

    src_v, dst_v = prep_edges(edge_index_void)
    src_s, dst_s = prep_edges(edge_index_vessel)

    zrow = jnp.zeros((ZR, CH), jnp.float32)
    zcnt = jnp.zeros((ZR, 16), jnp.float32)
    ones_h = jnp.ones((RW, 16), jnp.float32)
    b3_v = batch_void.reshape(GRID, 1, BN)
    b3_s = batch_vessel.reshape(GRID, 1, BN)

    cnt_v, cnt_s = _counts(dst_v, dst_s, ones_h, zcnt)

    s0_v, s0_s = _segsum(src_v, src_s, dst_v, dst_s,
                         x_void.reshape(NCH * N, CH),
                         x_vessel.reshape(NCH * N, CH), zrow)
    h0_v = _tc_layer0(s0_v, x_void, cnt_v, Wl0_void, Wr0_void,
                      jnp.tile(bl0_void[None, :], (8, 1)))
    h0_s = _tc_layer0(s0_s, x_vessel, cnt_s, Wl0_vessel, Wr0_vessel,
                      jnp.tile(bl0_vessel[None, :], (8, 1)))

    s1_v, s1_s = _segsum(src_v, src_s, dst_v, dst_s,
                         h0_v.reshape(NCH * N, CH),
                         h0_s.reshape(NCH * N, CH), zrow)
    pool_v, pcnt_v = _tc_layer1(s1_v, h0_v, cnt_v, Wl1_void, Wr1_void,
                                jnp.tile(bl1_void[None, :], (8, 1)), b3_v)
    pool_s, pcnt_s = _tc_layer1(s1_s, h0_s, cnt_s, Wl1_vessel, Wr1_vessel,
                                jnp.tile(bl1_vessel[None, :], (8, 1)), b3_s)

    out = pl.pallas_call(
        _head_body,
        out_shape=jax.ShapeDtypeStruct((G, lin_W.shape[1]), jnp.float32),
    )(pool_v, pcnt_v, pool_s, pcnt_s, lin_W,
      jnp.tile(lin_b[None, :], (8, 1)))
    return out
